# baseline (speedup 1.0000x reference)
import functools

import jax
import jax.numpy as jnp
from jax import lax
from jax.experimental import pallas as pl
from jax.experimental.pallas import tpu as pltpu
from jax.experimental.pallas import tpu_sc as plsc

EMB = 32
LANES = 128
ROWS_PER_LINE = LANES // EMB
IDX_CHUNK = 128


def _sc_gather_lines(uq2d, iq2d, t_ug, t_ig, t_um, t_im, batch):
    info = plsc.get_sparse_core_info()
    nc, ns = info.num_cores, info.num_subcores
    nw = nc * ns
    rows_per_w = batch // nw
    chunks = rows_per_w // IDX_CHUNK
    mesh = plsc.VectorSubcoreMesh(core_axis_name="c", subcore_axis_name="s")

    @functools.partial(
        pl.kernel,
        mesh=mesh,
        out_type=[jax.ShapeDtypeStruct((batch, LANES), jnp.float32)] * 4,
        scratch_types=[
            pltpu.VMEM((chunks, IDX_CHUNK), jnp.int32),
            pltpu.VMEM((chunks, IDX_CHUNK), jnp.int32),
            pltpu.VMEM((IDX_CHUNK, LANES), jnp.float32),
            pltpu.VMEM((IDX_CHUNK, LANES), jnp.float32),
            pltpu.SemaphoreType.DMA,
            pltpu.SemaphoreType.DMA,
            pltpu.SemaphoreType.DMA,
            pltpu.SemaphoreType.DMA,
        ],
        compiler_params=pltpu.CompilerParams(use_tc_tiling_on_sc=False),
    )
    def k(uq_hbm, iq_hbm, ug_hbm, ig_hbm, um_hbm, im_hbm,
          oug, oig, oum, oim, uv, iv, buf0, buf1,
          sem_g0, sem_g1, sem_w0, sem_w1):
        wid = lax.axis_index("s") * nc + lax.axis_index("c")
        crow = wid * chunks
        base = wid * rows_per_w
        pltpu.sync_copy(uq_hbm.at[pl.ds(crow, chunks)], uv)
        pltpu.sync_copy(iq_hbm.at[pl.ds(crow, chunks)], iv)
        bufs = (buf0, buf1)
        sems_g = (sem_g0, sem_g1)
        sems_w = (sem_w0, sem_w1)
        plan = []
        for tbl, out_hbm, idx in ((ug_hbm, oug, uv), (ig_hbm, oig, iv),
                                  (um_hbm, oum, uv), (im_hbm, oim, iv)):
            for j in range(chunks):
                plan.append((tbl, out_hbm, idx, j))
        n = len(plan)
        hs_g, hs_w = [None] * n, [None] * n
        for k_ in range(n):
            p = k_ % 2
            tbl, out_hbm, idx, j = plan[k_]
            if k_ >= 2:
                hs_w[k_ - 2].wait()
            hs_g[k_] = pltpu.async_copy(
                tbl.at[idx.at[j]], bufs[p], sems_g[p])
            if k_ >= 1:
                pm = (k_ - 1) % 2
                tblm, outm, idxm, jm = plan[k_ - 1]
                hs_g[k_ - 1].wait()
                hs_w[k_ - 1] = pltpu.async_copy(
                    bufs[pm],
                    outm.at[pl.ds(base + jm * IDX_CHUNK, IDX_CHUNK)],
                    sems_w[pm])
        tbl, out_hbm, idx, j = plan[n - 1]
        hs_g[n - 1].wait()
        hs_w[n - 1] = pltpu.async_copy(
            bufs[(n - 1) % 2],
            out_hbm.at[pl.ds(base + j * IDX_CHUNK, IDX_CHUNK)],
            sems_w[(n - 1) % 2])
        hs_w[n - 2].wait()
        hs_w[n - 1].wait()

    return k(uq2d, iq2d, t_ug, t_ig, t_um, t_im)


RELAYOUT_CL = 512


def _tc_relayout(tt, nblk, nlines):
    n = tt.shape[1]
    cl = RELAYOUT_CL
    last_blk = (n + cl - 1) // cl - 1
    nwin = LANES // EMB

    def body(i0, i1, i2, i3, out_ref):
        for m, r in enumerate((i0, i1, i2, i3)):
            out_ref[:, m * EMB:(m + 1) * EMB] = r[...].T

    def mk_map(m):
        return lambda i: (0, jnp.minimum(i + m * nblk, last_blk))

    return pl.pallas_call(
        body,
        grid=(nblk,),
        in_specs=[pl.BlockSpec((EMB, cl), mk_map(m))
                  for m in range(nwin)],
        out_specs=pl.BlockSpec((cl, LANES), lambda i: (i, 0)),
        out_shape=jax.ShapeDtypeStruct((nlines, LANES), jnp.float32),
    )(tt, tt, tt, tt)


def _tc_dense(gu_l, gi_l, mu_l, mi_l, urem, irem,
              w1u, w1i, b1, w2, b2, w3, b3, w4, b4, wpg, wph, bp):
    batch = gu_l.shape[0]
    nblk = 8
    blk = batch // nblk

    def extract(x, rem):
        y = jnp.zeros((x.shape[0], EMB), jnp.float32)
        for m in range(ROWS_PER_LINE):
            y = y + jnp.where(rem == m, x[:, m * EMB:(m + 1) * EMB], 0.0)
        return y

    def body(gu_ref, gi_ref, mu_ref, mi_ref, urem_ref, irem_ref,
             w1u_ref, w1i_ref, b1_ref, w2_ref, b2_ref, w3_ref, b3_ref,
             w4_ref, b4_ref, wpg_ref, wph_ref, bp_ref, out_ref):
        ur = urem_ref[...]
        ir = irem_ref[...]
        gu = extract(gu_ref[...], ur)
        gi = extract(gi_ref[...], ir)
        mu = extract(mu_ref[...], ur)
        mi = extract(mi_ref[...], ir)
        dg = lambda x, w: lax.dot_general(
            x, w, (((1,), (1,)), ((), ())),
            preferred_element_type=jnp.float32)
        h = jnp.maximum(dg(mu, w1u_ref[...])
                        + dg(mi, w1i_ref[...]) + b1_ref[...], 0.0)
        h = jnp.maximum(dg(h, w2_ref[...]) + b2_ref[...], 0.0)
        h = jnp.maximum(dg(h, w3_ref[...]) + b3_ref[...], 0.0)
        h = jnp.maximum(dg(h, w4_ref[...]) + b4_ref[...], 0.0)
        g = gu * gi
        pred = (jnp.sum(g * wpg_ref[...], axis=1)
                + jnp.sum(h * wph_ref[...], axis=1) + bp_ref[0, 0])
        out_ref[...] = jax.nn.sigmoid(pred)

    data_spec = pl.BlockSpec((blk, LANES), lambda i: (i, 0))
    rem_spec = pl.BlockSpec((blk, 1), lambda i: (i, 0))
    full = lambda a: pl.BlockSpec(a.shape, lambda i: tuple(0 for _ in a.shape))
    return pl.pallas_call(
        body,
        grid=(nblk,),
        in_specs=[data_spec] * 4 + [rem_spec] * 2
        + [full(w) for w in (w1u, w1i, b1, w2, b2, w3, b3, w4, b4,
                             wpg, wph, bp)],
        out_specs=pl.BlockSpec((blk,), lambda i: (i,)),
        out_shape=jax.ShapeDtypeStruct((batch,), jnp.float32),
    )(gu_l, gi_l, mu_l, mi_l, urem, irem,
      w1u, w1i, b1, w2, b2, w3, b3, w4, b4, wpg, wph, bp)


def kernel(user_indices, item_indices, emb_user_gmf, emb_item_gmf,
           emb_user_mlp, emb_item_mlp, W1, b1, W2, b2, W3, b3, W4, b4,
           Wp, bp):
    batch = user_indices.shape[0]
    ui = user_indices.astype(jnp.int32)
    ii = item_indices.astype(jnp.int32)
    n = emb_user_gmf.shape[0]
    nblk = (n + 4 * RELAYOUT_CL - 1) // (4 * RELAYOUT_CL)
    nlines = RELAYOUT_CL * nblk
    uq2d = (ui % nlines).reshape(batch // IDX_CHUNK, IDX_CHUNK)
    iq2d = (ii % nlines).reshape(batch // IDX_CHUNK, IDX_CHUNK)
    urem = (ui // nlines).reshape(batch, 1)
    irem = (ii // nlines).reshape(batch, 1)
    lines = [_tc_relayout(t.T, nblk, nlines)
             for t in (emb_user_gmf, emb_item_gmf,
                       emb_user_mlp, emb_item_mlp)]
    gu_l, gi_l, mu_l, mi_l = _sc_gather_lines(uq2d, iq2d, *lines, batch)
    return _tc_dense(
        gu_l, gi_l, mu_l, mi_l, urem, irem,
        W1[:, :EMB], W1[:, EMB:], b1.reshape(1, -1),
        W2, b2.reshape(1, -1), W3, b3.reshape(1, -1),
        W4, b4.reshape(1, -1),
        Wp[:, :EMB], Wp[:, EMB:], bp.reshape(1, 1))


# speedup vs baseline: 1.5440x; 1.5440x over previous
import functools

import jax
import jax.numpy as jnp
from jax import lax
from jax.experimental import pallas as pl
from jax.experimental.pallas import tpu as pltpu
from jax.experimental.pallas import tpu_sc as plsc

EMB = 32
LANES = 128
ROWS_PER_LINE = LANES // EMB
IDX_CHUNK = 128


def _sc_gather_lines(uq2d, iq2d, t_ug, t_ig, t_um, t_im, batch):
    info = plsc.get_sparse_core_info()
    nc, ns = info.num_cores, info.num_subcores
    nw = nc * ns
    rows_per_w = batch // nw
    chunks = rows_per_w // IDX_CHUNK
    mesh = plsc.VectorSubcoreMesh(core_axis_name="c", subcore_axis_name="s")

    @functools.partial(
        pl.kernel,
        mesh=mesh,
        out_type=[jax.ShapeDtypeStruct((batch, LANES), jnp.float32)] * 4,
        scratch_types=[
            pltpu.VMEM((chunks, IDX_CHUNK), jnp.int32),
            pltpu.VMEM((chunks, IDX_CHUNK), jnp.int32),
            pltpu.VMEM((IDX_CHUNK, LANES), jnp.float32),
            pltpu.VMEM((IDX_CHUNK, LANES), jnp.float32),
            pltpu.SemaphoreType.DMA,
            pltpu.SemaphoreType.DMA,
            pltpu.SemaphoreType.DMA,
            pltpu.SemaphoreType.DMA,
        ],
        compiler_params=pltpu.CompilerParams(use_tc_tiling_on_sc=False),
    )
    def k(uq_hbm, iq_hbm, ug_hbm, ig_hbm, um_hbm, im_hbm,
          oug, oig, oum, oim, uv, iv, buf0, buf1,
          sem_g0, sem_g1, sem_w0, sem_w1):
        wid = lax.axis_index("s") * nc + lax.axis_index("c")
        crow = wid * chunks
        base = wid * rows_per_w
        pltpu.sync_copy(uq_hbm.at[pl.ds(crow, chunks)], uv)
        pltpu.sync_copy(iq_hbm.at[pl.ds(crow, chunks)], iv)
        bufs = (buf0, buf1)
        sems_g = (sem_g0, sem_g1)
        sems_w = (sem_w0, sem_w1)
        plan = []
        for tbl, out_hbm, idx in ((ug_hbm, oug, uv), (ig_hbm, oig, iv),
                                  (um_hbm, oum, uv), (im_hbm, oim, iv)):
            for j in range(chunks):
                plan.append((tbl, out_hbm, idx, j))
        n = len(plan)
        hs_g, hs_w = [None] * n, [None] * n
        for k_ in range(n):
            p = k_ % 2
            tbl, out_hbm, idx, j = plan[k_]
            if k_ >= 2:
                hs_w[k_ - 2].wait()
            hs_g[k_] = pltpu.async_copy(
                tbl.at[idx.at[j]], bufs[p], sems_g[p])
            if k_ >= 1:
                pm = (k_ - 1) % 2
                tblm, outm, idxm, jm = plan[k_ - 1]
                hs_g[k_ - 1].wait()
                hs_w[k_ - 1] = pltpu.async_copy(
                    bufs[pm],
                    outm.at[pl.ds(base + jm * IDX_CHUNK, IDX_CHUNK)],
                    sems_w[pm])
        tbl, out_hbm, idx, j = plan[n - 1]
        hs_g[n - 1].wait()
        hs_w[n - 1] = pltpu.async_copy(
            bufs[(n - 1) % 2],
            out_hbm.at[pl.ds(base + j * IDX_CHUNK, IDX_CHUNK)],
            sems_w[(n - 1) % 2])
        hs_w[n - 2].wait()
        hs_w[n - 1].wait()

    return k(uq2d, iq2d, t_ug, t_ig, t_um, t_im)


RELAYOUT_CL = 2048


def _tc_relayout(tt, nblk, nlines):
    n = tt.shape[1]
    cl = RELAYOUT_CL
    last_blk = (n + cl - 1) // cl - 1
    nwin = LANES // EMB

    def body(i0, i1, i2, i3, eye_ref, out_ref):
        eye = eye_ref[...]
        ys = [lax.dot_general(r[...], eye, (((0,), (0,)), ((), ())),
                              preferred_element_type=jnp.float32)
              for r in (i0, i1, i2, i3)]
        out_ref[...] = jnp.concatenate(ys, axis=1)

    def mk_map(m):
        return lambda i: (0, jnp.minimum(i + m * nblk, last_blk))

    eye = jnp.eye(EMB, dtype=jnp.float32)
    return pl.pallas_call(
        body,
        grid=(nblk,),
        in_specs=[pl.BlockSpec((EMB, cl), mk_map(m))
                  for m in range(nwin)]
        + [pl.BlockSpec((EMB, EMB), lambda i: (0, 0))],
        out_specs=pl.BlockSpec((cl, LANES), lambda i: (i, 0)),
        out_shape=jax.ShapeDtypeStruct((nlines, LANES), jnp.float32),
        compiler_params=pltpu.CompilerParams(
            fuse_transposed_lhs_in_matmul=True),
    )(tt, tt, tt, tt, eye)


def _tc_dense(gu_l, gi_l, mu_l, mi_l, urem, irem,
              w1u, w1i, b1, w2, b2, w3, b3, w4, b4, wpg, wph, bp):
    batch = gu_l.shape[0]
    nblk = 8
    blk = batch // nblk

    def extract(x, rem):
        y = jnp.zeros((x.shape[0], EMB), jnp.float32)
        for m in range(ROWS_PER_LINE):
            y = y + jnp.where(rem == m, x[:, m * EMB:(m + 1) * EMB], 0.0)
        return y

    def body(gu_ref, gi_ref, mu_ref, mi_ref, urem_ref, irem_ref,
             w1u_ref, w1i_ref, b1_ref, w2_ref, b2_ref, w3_ref, b3_ref,
             w4_ref, b4_ref, wpg_ref, wph_ref, bp_ref, out_ref):
        ur = urem_ref[...]
        ir = irem_ref[...]
        gu = extract(gu_ref[...], ur)
        gi = extract(gi_ref[...], ir)
        mu = extract(mu_ref[...], ur)
        mi = extract(mi_ref[...], ir)
        dg = lambda x, w: lax.dot_general(
            x, w, (((1,), (1,)), ((), ())),
            preferred_element_type=jnp.float32)
        h = jnp.maximum(dg(mu, w1u_ref[...])
                        + dg(mi, w1i_ref[...]) + b1_ref[...], 0.0)
        h = jnp.maximum(dg(h, w2_ref[...]) + b2_ref[...], 0.0)
        h = jnp.maximum(dg(h, w3_ref[...]) + b3_ref[...], 0.0)
        h = jnp.maximum(dg(h, w4_ref[...]) + b4_ref[...], 0.0)
        g = gu * gi
        pred = (jnp.sum(g * wpg_ref[...], axis=1)
                + jnp.sum(h * wph_ref[...], axis=1) + bp_ref[0, 0])
        out_ref[...] = jax.nn.sigmoid(pred)

    data_spec = pl.BlockSpec((blk, LANES), lambda i: (i, 0))
    rem_spec = pl.BlockSpec((blk, 1), lambda i: (i, 0))
    full = lambda a: pl.BlockSpec(a.shape, lambda i: tuple(0 for _ in a.shape))
    return pl.pallas_call(
        body,
        grid=(nblk,),
        in_specs=[data_spec] * 4 + [rem_spec] * 2
        + [full(w) for w in (w1u, w1i, b1, w2, b2, w3, b3, w4, b4,
                             wpg, wph, bp)],
        out_specs=pl.BlockSpec((blk,), lambda i: (i,)),
        out_shape=jax.ShapeDtypeStruct((batch,), jnp.float32),
    )(gu_l, gi_l, mu_l, mi_l, urem, irem,
      w1u, w1i, b1, w2, b2, w3, b3, w4, b4, wpg, wph, bp)


def kernel(user_indices, item_indices, emb_user_gmf, emb_item_gmf,
           emb_user_mlp, emb_item_mlp, W1, b1, W2, b2, W3, b3, W4, b4,
           Wp, bp):
    batch = user_indices.shape[0]
    ui = user_indices.astype(jnp.int32)
    ii = item_indices.astype(jnp.int32)
    n = emb_user_gmf.shape[0]
    nblk = (n + 4 * RELAYOUT_CL - 1) // (4 * RELAYOUT_CL)
    nlines = RELAYOUT_CL * nblk
    uq2d = (ui % nlines).reshape(batch // IDX_CHUNK, IDX_CHUNK)
    iq2d = (ii % nlines).reshape(batch // IDX_CHUNK, IDX_CHUNK)
    urem = (ui // nlines).reshape(batch, 1)
    irem = (ii // nlines).reshape(batch, 1)
    lines = [_tc_relayout(t.T, nblk, nlines)
             for t in (emb_user_gmf, emb_item_gmf,
                       emb_user_mlp, emb_item_mlp)]
    gu_l, gi_l, mu_l, mi_l = _sc_gather_lines(uq2d, iq2d, *lines, batch)
    return _tc_dense(
        gu_l, gi_l, mu_l, mi_l, urem, irem,
        W1[:, :EMB], W1[:, EMB:], b1.reshape(1, -1),
        W2, b2.reshape(1, -1), W3, b3.reshape(1, -1),
        W4, b4.reshape(1, -1),
        Wp[:, :EMB], Wp[:, EMB:], bp.reshape(1, 1))
